# baseline (speedup 1.0000x reference)
import functools

import numpy as np
import jax
import jax.numpy as jnp
from jax import lax
from jax.experimental import pallas as pl
from jax.experimental.pallas import tpu as pltpu
from jax.experimental.pallas import tpu_sc as plsc

_NB_POOLS = 256
_LANES = 16


def _batcher_stages(nb):
    stages = []
    p = 1
    while p < nb:
        k = p
        while k >= 1:
            nj = len(range(k % p, nb - k, 2 * k))
            if nj > 0:
                stages.append((p, k, nj))
            k //= 2
        p *= 2
    return stages


def _corr_pallas(x3):
    B, M, F = x3.shape

    def body(x_ref, o_ref):
        a = x_ref[0]
        c = lax.dot_general(
            a, a, (((1,), (1,)), ((), ())),
            preferred_element_type=jnp.float32,
            precision=lax.Precision.HIGHEST,
        )
        o_ref[0] = c * (1.0 / F)

    return pl.pallas_call(
        body,
        grid=(B,),
        in_specs=[pl.BlockSpec((1, M, F), lambda b: (b, 0, 0))],
        out_specs=pl.BlockSpec((1, M, M), lambda b: (b, 0, 0)),
        out_shape=jax.ShapeDtypeStruct((B, M, M), jnp.float32),
    )(x3)


def _sc_sort_pool(corr2, asc_idx):
    R, M = corr2.shape
    NB = M // _LANES
    NP = asc_idx.shape[0]
    info = plsc.get_sparse_core_info()
    NC, NS = info.num_cores, info.num_subcores
    NW = NC * NS
    rpw = R // NW
    CH = 8
    stages = _batcher_stages(NB)
    mesh = plsc.VectorSubcoreMesh(core_axis_name="c", subcore_axis_name="s")

    @functools.partial(
        pl.kernel,
        mesh=mesh,
        compiler_params=pltpu.CompilerParams(needs_layout_passes=False),
        out_type=jax.ShapeDtypeStruct((R, NP), jnp.float32),
        scratch_types=[
            pltpu.VMEM((CH, M), jnp.float32),
            pltpu.VMEM((CH, NP), jnp.float32),
            pltpu.VMEM((NP,), jnp.int32),
        ],
    )
    def k(corr_hbm, idx_hbm, out_hbm, buf, obuf, idxv):
        wid = lax.axis_index("s") * NC + lax.axis_index("c")
        pltpu.sync_copy(idx_hbm, idxv)

        def chunk(ci, carry):
            base = wid * rpw + ci * CH
            pltpu.sync_copy(corr_hbm.at[pl.ds(base, CH)], buf)

            def presort(b, c2):
                for r in range(CH):
                    v = buf[r, pl.ds(b * _LANES, _LANES)]
                    buf[r, pl.ds(b * _LANES, _LANES)] = jnp.sort(v)
                return c2

            lax.fori_loop(0, NB, presort, 0)

            for (p, kk, nj) in stages:
                def merge_step(t, c2, p=p, kk=kk):
                    tj = t // kk
                    ti = t - tj * kk
                    m = (kk % p) + tj * (2 * kk) + ti
                    valid = jnp.logical_and(
                        (m // (2 * p)) == ((m + kk) // (2 * p)),
                        (m + kk) < NB,
                    )

                    @pl.when(valid)
                    def _():
                        a0 = m * _LANES
                        b0 = (m + kk) * _LANES
                        for r in range(CH):
                            va = buf[r, pl.ds(a0, _LANES)]
                            vb = buf[r, pl.ds(b0, _LANES)]
                            rb = lax.rev(vb, (0,))
                            lo = jnp.minimum(va, rb)
                            hi = jnp.maximum(va, rb)
                            buf[r, pl.ds(a0, _LANES)] = jnp.sort(lo)
                            buf[r, pl.ds(b0, _LANES)] = jnp.sort(hi)

                    return c2

                lax.fori_loop(0, nj * kk, merge_step, 0)

            def gather(g, c2):
                iv = idxv[pl.ds(g * _LANES, _LANES)]
                for r in range(CH):
                    ridx = jnp.full((_LANES,), r, jnp.int32)
                    vals = plsc.load_gather(buf, [ridx, iv])
                    obuf[r, pl.ds(g * _LANES, _LANES)] = vals
                return c2

            lax.fori_loop(0, NP // _LANES, gather, 0)

            pltpu.sync_copy(obuf, out_hbm.at[pl.ds(base, CH)])
            return carry

        lax.fori_loop(0, rpw // CH, chunk, 0)

    return k(corr2, asc_idx)


def kernel(x):
    B, H, W, F = x.shape
    M = H * W
    x3 = x.reshape(B, M, F)
    corr = _corr_pallas(x3)
    corr2 = corr.reshape(B * M, M)
    ranks = np.round(np.linspace(1.0, M - 1, _NB_POOLS)).astype(np.int64)
    asc = jnp.asarray((M - 1 - ranks).astype(np.int32))
    out = _sc_sort_pool(corr2, asc)
    return out.reshape(B, H, W, _NB_POOLS)
